# Initial kernel scaffold; baseline (speedup 1.0000x reference)
import jax
import jax.numpy as jnp
from jax.experimental import pallas as pl


def kernel(x, edges0, edges1, Wc10, bc10, Wc1l, bc1l, Wc20, bc20, Wc2l, bc2l, AW0, AW1, a_att, r_att):
    raise NotImplementedError("write your pallas kernel here")


# speedup vs baseline: 1.3084x; 1.3084x over previous
import jax
import jax.numpy as jnp
import numpy as np
from jax import lax
from jax.experimental import pallas as pl
from jax.experimental.pallas import tpu as pltpu
from jax.experimental.pallas import tpu_sc as plsc

N = 10000
D = 512
E = 75000

NPAD = 10240
RB = 256
NBLK = NPAD // RB
CH = 128
SUBS = D // CH
NMAT = 3
NPART = 2 * NMAT * SUBS + 2

NTILE = 16
NCORE = 2
NQ = 4
QROWS = NPAD // NQ
ACC_ROWS = 2816
TRASH = QROWS
ZQ_PT = ACC_ROWS // NTILE
DQ_PT = QROWS // NTILE

EB = 128
EPT = 4688
NRW = EPT // 16
CAP = 4736
NB_E = CAP // EB
NFB = CAP // 16

_f32 = jnp.float32
_i32 = jnp.int32


def _prep_a_body(x_ref, w0_ref, w1_ref, w2_ref, b0_ref, b1_ref, out_ref):
    m = pl.program_id(1)

    def write(v):
        for sub in range(SUBS):
            out_ref[:, sub, :] = v[:, sub * CH:(sub + 1) * CH]

    @pl.when(m == 0)
    def _m0():
        write(x_ref[...])

    @pl.when(m == 1)
    def _m1():
        write(jnp.dot(x_ref[...], w0_ref[...],
                      preferred_element_type=_f32) + b0_ref[...])

    @pl.when(m == 2)
    def _m2():
        write(jnp.dot(x_ref[...], w1_ref[...],
                      preferred_element_type=_f32) + b1_ref[...])

    @pl.when(m == 3)
    def _m3():
        write(jnp.dot(x_ref[...], w2_ref[...], preferred_element_type=_f32))


def _tc_prep_a(x, w0, w1, w2, b0, b1):
    return pl.pallas_call(
        _prep_a_body,
        grid=(NBLK, 4),
        in_specs=[
            pl.BlockSpec((RB, D), lambda i, m: (i, 0)),
            pl.BlockSpec((D, D), lambda i, m: (0, 0)),
            pl.BlockSpec((D, D), lambda i, m: (0, 0)),
            pl.BlockSpec((D, D), lambda i, m: (0, 0)),
            pl.BlockSpec((1, D), lambda i, m: (0, 0)),
            pl.BlockSpec((1, D), lambda i, m: (0, 0)),
        ],
        out_specs=pl.BlockSpec((RB, SUBS, CH),
                               lambda i, m: (m * NBLK + i, 0, 0)),
        out_shape=jax.ShapeDtypeStruct((4 * NPAD, SUBS, CH), _f32),
    )(x, w0, w1, w2, b0, b1)


def _prep_r_body(x1_ref, x2_ref, w0_ref, w1_ref, w2_ref, b0_ref, b1_ref,
                 out_ref):
    m = pl.program_id(1)

    def write(v):
        for sub in range(SUBS):
            out_ref[:, sub, :] = v[:, sub * CH:(sub + 1) * CH]

    def mm(x3_ref, w_ref):
        acc = jnp.zeros((RB, D), _f32)
        for s in range(SUBS):
            acc = acc + jnp.dot(x3_ref[:, s, :], w_ref[s],
                                preferred_element_type=_f32)
        return acc

    @pl.when(m == 0)
    def _m0():
        for sub in range(SUBS):
            out_ref[:, sub, :] = x1_ref[:, sub, :]

    @pl.when(m == 1)
    def _m1():
        write(mm(x1_ref, w0_ref) + b0_ref[...])

    @pl.when(m == 2)
    def _m2():
        write(mm(x2_ref, w1_ref) + b1_ref[...])

    @pl.when(m == 3)
    def _m3():
        write(mm(x1_ref, w2_ref))


def _tc_prep_r(x1, x2, w0, w1, w2, b0, b1):
    return pl.pallas_call(
        _prep_r_body,
        grid=(NBLK, 4),
        in_specs=[
            pl.BlockSpec((RB, SUBS, CH), lambda i, m: (i, 0, 0)),
            pl.BlockSpec((RB, SUBS, CH), lambda i, m: (i, 0, 0)),
            pl.BlockSpec((SUBS, CH, D), lambda i, m: (0, 0, 0)),
            pl.BlockSpec((SUBS, CH, D), lambda i, m: (0, 0, 0)),
            pl.BlockSpec((SUBS, CH, D), lambda i, m: (0, 0, 0)),
            pl.BlockSpec((1, D), lambda i, m: (0, 0)),
            pl.BlockSpec((1, D), lambda i, m: (0, 0)),
        ],
        out_specs=pl.BlockSpec((RB, SUBS, CH),
                               lambda i, m: (m * NBLK + i, 0, 0)),
        out_shape=jax.ShapeDtypeStruct((4 * NPAD, SUBS, CH), _f32),
    )(x1, x2, w0, w1, w2, b0, b1)


_INV_SQRT_D = 1.0 / np.sqrt(float(D))


def _epi_common(p_ref, xaw_ref, att_ref):
    t = att_ref[0] - att_ref[1]
    att0 = jax.nn.sigmoid(jnp.full((RB, 1), t, _f32))
    att1 = 1.0 - att0
    deg0 = p_ref[2 * NMAT * SUBS, :, 0:1]
    deg1 = p_ref[2 * NMAT * SUBS + 1, :, 0:1]
    s0 = att0 / jnp.maximum(deg0, 1.0)
    s1 = att1 / jnp.maximum(deg1, 1.0)
    comb = [[s0 * p_ref[(0 * NMAT + mat) * SUBS + sub]
             + s1 * p_ref[(1 * NMAT + mat) * SUBS + sub]
             for sub in range(SUBS)] for mat in range(NMAT)]
    logits = jnp.zeros((RB, 1), _f32)
    for sub in range(SUBS):
        logits = logits + jnp.sum(comb[0][sub] * xaw_ref[0, :, sub, :],
                                  axis=1, keepdims=True)
    attw = jax.nn.sigmoid(logits * _INV_SQRT_D)
    return comb, attw


def _epi_a_body(p_ref, s2_ref, xaw_ref, att_ref, x1_ref, x2_ref):
    comb, attw = _epi_common(p_ref, xaw_ref, att_ref)
    for sub in range(SUBS):
        a1 = comb[1][sub]
        y1 = jnp.where(a1 >= 0, a1, 0.2 * a1)
        t2 = s2_ref[0, :, sub, :] - comb[2][sub]
        y2 = jnp.where(t2 >= 0, t2, 0.2 * t2)
        x1_ref[:, sub, :] = y1 + (1.0 - attw) * y2
        x2_ref[:, sub, :] = y2 + attw * y1


def _tc_epi_a(parts, catg4, att):
    return pl.pallas_call(
        _epi_a_body,
        grid=(NBLK,),
        in_specs=[
            pl.BlockSpec((NPART, RB, CH), lambda i: (0, i, 0)),
            pl.BlockSpec((1, RB, SUBS, CH), lambda i: (2, i, 0, 0)),
            pl.BlockSpec((1, RB, SUBS, CH), lambda i: (3, i, 0, 0)),
            pl.BlockSpec(memory_space=pltpu.SMEM),
        ],
        out_specs=[
            pl.BlockSpec((RB, SUBS, CH), lambda i: (i, 0, 0)),
            pl.BlockSpec((RB, SUBS, CH), lambda i: (i, 0, 0)),
        ],
        out_shape=[
            jax.ShapeDtypeStruct((NPAD, SUBS, CH), _f32),
            jax.ShapeDtypeStruct((NPAD, SUBS, CH), _f32),
        ],
    )(parts, catg4, catg4, att)


def _epi_r_body(p_ref, s2_ref, xaw_ref, att_ref, out_ref):
    comb, attw = _epi_common(p_ref, xaw_ref, att_ref)
    o = []
    ss = jnp.zeros((RB, 1), _f32)
    for sub in range(SUBS):
        o_sub = comb[1][sub] + (1.0 - attw) * (s2_ref[0, :, sub, :]
                                               - comb[2][sub])
        o.append(o_sub)
        ss = ss + jnp.sum(o_sub * o_sub, axis=1, keepdims=True)
    inv = 1.0 / jnp.maximum(jnp.sqrt(ss), 1e-12)
    for sub in range(SUBS):
        out_ref[:, sub * CH:(sub + 1) * CH] = o[sub] * inv


def _tc_epi_r(parts, catg4, att):
    return pl.pallas_call(
        _epi_r_body,
        grid=(NBLK,),
        in_specs=[
            pl.BlockSpec((NPART, RB, CH), lambda i: (0, i, 0)),
            pl.BlockSpec((1, RB, SUBS, CH), lambda i: (2, i, 0, 0)),
            pl.BlockSpec((1, RB, SUBS, CH), lambda i: (3, i, 0, 0)),
            pl.BlockSpec(memory_space=pltpu.SMEM),
        ],
        out_specs=pl.BlockSpec((RB, D), lambda i: (i, 0)),
        out_shape=jax.ShapeDtypeStruct((NPAD, D), _f32),
    )(parts, catg4, catg4, att)


def _sc_body(catv_hbm, src0_hbm, dst0_hbm, src1_hbm, dst1_hbm,
             p_hbm,
             sflat, dflat, c2ds, c2dd,
             zbuf_v, gidx_v, gbuf_v, obuf_v, gsem,
             acc_sh):
    cid = lax.axis_index("c")
    sid = lax.axis_index("s")

    start = jnp.minimum(sid * EPT, E - EPT)
    delta = sid * EPT - start
    ramp16 = lax.iota(_i32, 16)
    zero16f = jnp.zeros((16,), _f32)
    zero16i = jnp.zeros((16,), _i32)
    trash16 = jnp.full((16,), TRASH, _i32)
    ones16 = jnp.full((16,), 1.0, _f32)

    @pl.when(cid == 0)
    def _l0():
        pltpu.sync_copy(src0_hbm.at[pl.ds(start, EPT)], sflat)
        pltpu.sync_copy(dst0_hbm.at[pl.ds(start, EPT)], dflat)

    @pl.when(cid == 1)
    def _l1():
        pltpu.sync_copy(src1_hbm.at[pl.ds(start, EPT)], sflat)
        pltpu.sync_copy(dst1_hbm.at[pl.ds(start, EPT)], dflat)

    @pl.loop(0, ZQ_PT)
    def _zb(r):
        for c in range(CH // 16):
            zbuf_v[r, pl.ds(c * 16, 16)] = zero16f

    @pl.loop(0, EB)
    def _ob(r):
        for c in range(CH // 16):
            gbuf_v[1, r, pl.ds(c * 16, 16)] = ones16

    def zero_acc():
        pltpu.sync_copy(zbuf_v, acc_sh.at[pl.ds(sid * ZQ_PT, ZQ_PT)])

    def drain(prow):
        pltpu.sync_copy(acc_sh.at[pl.ds(sid * DQ_PT, DQ_PT)], obuf_v)
        pltpu.sync_copy(obuf_v, p_hbm.at[pl.ds(prow + sid * DQ_PT, DQ_PT)])

    @pl.loop(0, NRW)
    def _s2d(b):
        r = b // 8
        c = (b - r * 8) * 16
        c2ds[r, pl.ds(c, 16)] = sflat[pl.ds(b * 16, 16)]
    for k in range(NRW, NFB):
        r = k // 8
        c = (k - r * 8) * 16
        c2ds[r, pl.ds(c, 16)] = zero16i
        c2dd[r, pl.ds(c, 16)] = trash16

    for h in range(NQ):
        lo = h * QROWS

        @pl.loop(0, NRW)
        def _d2d(b, _lo=lo):
            dv = dflat[pl.ds(b * 16, 16)]
            pos = ramp16 + b * 16
            m = (pos >= delta) & (dv >= _lo) & (dv < _lo + QROWS)
            r = b // 8
            c = (b - r * 8) * 16
            c2dd[r, pl.ds(c, 16)] = jnp.where(m, dv - _lo, TRASH)

        nb = NB_E

        def edge_pass(gather_base, _nb=nb):
            @pl.loop(0, _nb)
            def _edges(j):
                for i in range(EB // 16):
                    sl = pl.ds(i * 16, 16)
                    gidx_v[0, sl] = c2ds[j, sl] * SUBS + gather_base
                pltpu.async_copy(catv_hbm.at[gidx_v.at[0]], gbuf_v.at[0],
                                 gsem).wait()
                pltpu.sync_copy(gbuf_v.at[0], acc_sh.at[c2dd.at[j]],
                                add=True)

        @pl.loop(0, NMAT * SUBS)
        def _chunk(fc, _lo=lo, _ep=edge_pass):
            mat = fc // SUBS
            sub = fc - mat * SUBS
            zero_acc()
            plsc.subcore_barrier()
            _ep(mat * (NPAD * SUBS) + sub)
            plsc.subcore_barrier()
            drain(((cid * NMAT + mat) * SUBS + sub) * NPAD + _lo)

        zero_acc()
        plsc.subcore_barrier()

        @pl.loop(0, nb)
        def _deg(j):
            pltpu.sync_copy(gbuf_v.at[1], acc_sh.at[c2dd.at[j]], add=True)

        plsc.subcore_barrier()
        drain((2 * NMAT * SUBS + cid) * NPAD + lo)


def _sc_spmm(catv, src0, dst0, src1, dst1):
    mesh = plsc.VectorSubcoreMesh(core_axis_name="c", subcore_axis_name="s",
                                  num_cores=NCORE, num_subcores=NTILE)
    f = pl.kernel(
        _sc_body,
        out_type=jax.ShapeDtypeStruct((NPART * NPAD, CH), _f32),
        mesh=mesh,
        scratch_types=[
            pltpu.VMEM((EPT,), _i32),
            pltpu.VMEM((EPT,), _i32),
            pltpu.VMEM((NB_E, EB), _i32),
            pltpu.VMEM((NB_E, EB), _i32),
            pltpu.VMEM((ZQ_PT, CH), _f32),
            pltpu.VMEM((1, EB), _i32),
            pltpu.VMEM((2, EB, CH), _f32),
            pltpu.VMEM((DQ_PT, CH), _f32),
            pltpu.SemaphoreType.DMA,
            pltpu.VMEM_SHARED((ACC_ROWS, CH), _f32),
        ],
    )
    return f(catv, src0, dst0, src1, dst1)


def kernel(x, edges0, edges1, Wc10, bc10, Wc1l, bc1l, Wc20, bc20, Wc2l, bc2l,
           AW0, AW1, a_att, r_att):
    e0c0 = edges0[:, 0]
    e0c1 = edges0[:, 1]
    e1c0 = edges1[:, 0]
    e1c1 = edges1[:, 1]

    catg_a = _tc_prep_a(x, Wc10, Wc20, AW0,
                        bc10.reshape(1, D), bc20.reshape(1, D))
    parts_a = _sc_spmm(catg_a.reshape(4 * NPAD * SUBS, CH),
                       e0c1, e0c0, e1c1, e1c0)
    x1, x2 = _tc_epi_a(parts_a.reshape(NPART, NPAD, CH),
                       catg_a.reshape(4, NPAD, SUBS, CH), a_att)

    catg_r = _tc_prep_r(x1, x2,
                        Wc1l.reshape(SUBS, CH, D), Wc2l.reshape(SUBS, CH, D),
                        AW1.reshape(SUBS, CH, D),
                        bc1l.reshape(1, D), bc2l.reshape(1, D))
    parts_r = _sc_spmm(catg_r.reshape(4 * NPAD * SUBS, CH),
                       e0c0, e0c1, e1c0, e1c1)
    out = _tc_epi_r(parts_r.reshape(NPART, NPAD, CH),
                    catg_r.reshape(4, NPAD, SUBS, CH), r_att)
    return out[:N]


# speedup vs baseline: 1.3614x; 1.0405x over previous
import jax
import jax.numpy as jnp
import numpy as np
from jax import lax
from jax.experimental import pallas as pl
from jax.experimental.pallas import tpu as pltpu
from jax.experimental.pallas import tpu_sc as plsc

N = 10000
D = 512
E = 75000

NPAD = 10240
RB = 256
NBLK = NPAD // RB
CH = 128
SUBS = D // CH
NMAT = 3
NPART = 2 * NMAT * SUBS + 2

NTILE = 16
NCORE = 2
NQ = 4
QROWS = 2560
ACC_ROWS = 2816
TRASH = QROWS
ZQ_PT = ACC_ROWS // NTILE

EB = 128
EPT = 4688
NRW = EPT // 16
CAP = 4736
NB_E = CAP // EB
NFB = CAP // 16

_f32 = jnp.float32
_i32 = jnp.int32


def _prep_a_body(x_ref, w0_ref, w1_ref, w2_ref, b0_ref, b1_ref, out_ref):
    m = pl.program_id(1)

    def write(v):
        for sub in range(SUBS):
            out_ref[:, sub, :] = v[:, sub * CH:(sub + 1) * CH]

    @pl.when(m == 0)
    def _m0():
        write(x_ref[...])

    @pl.when(m == 1)
    def _m1():
        write(jnp.dot(x_ref[...], w0_ref[...],
                      preferred_element_type=_f32) + b0_ref[...])

    @pl.when(m == 2)
    def _m2():
        write(jnp.dot(x_ref[...], w1_ref[...],
                      preferred_element_type=_f32) + b1_ref[...])

    @pl.when(m == 3)
    def _m3():
        write(jnp.dot(x_ref[...], w2_ref[...], preferred_element_type=_f32))


def _tc_prep_a(x, w0, w1, w2, b0, b1):
    return pl.pallas_call(
        _prep_a_body,
        grid=(NBLK, 4),
        in_specs=[
            pl.BlockSpec((RB, D), lambda i, m: (i, 0)),
            pl.BlockSpec((D, D), lambda i, m: (0, 0)),
            pl.BlockSpec((D, D), lambda i, m: (0, 0)),
            pl.BlockSpec((D, D), lambda i, m: (0, 0)),
            pl.BlockSpec((1, D), lambda i, m: (0, 0)),
            pl.BlockSpec((1, D), lambda i, m: (0, 0)),
        ],
        out_specs=pl.BlockSpec((RB, SUBS, CH),
                               lambda i, m: (m * NBLK + i, 0, 0)),
        out_shape=jax.ShapeDtypeStruct((4 * NPAD, SUBS, CH), _f32),
    )(x, w0, w1, w2, b0, b1)


def _prep_r_body(x1_ref, x2_ref, w0_ref, w1_ref, w2_ref, b0_ref, b1_ref,
                 out_ref):
    m = pl.program_id(1)

    def write(v):
        for sub in range(SUBS):
            out_ref[:, sub, :] = v[:, sub * CH:(sub + 1) * CH]

    def mm(x3_ref, w_ref):
        acc = jnp.zeros((RB, D), _f32)
        for s in range(SUBS):
            acc = acc + jnp.dot(x3_ref[:, s, :], w_ref[s],
                                preferred_element_type=_f32)
        return acc

    @pl.when(m == 0)
    def _m0():
        for sub in range(SUBS):
            out_ref[:, sub, :] = x1_ref[:, sub, :]

    @pl.when(m == 1)
    def _m1():
        write(mm(x1_ref, w0_ref) + b0_ref[...])

    @pl.when(m == 2)
    def _m2():
        write(mm(x2_ref, w1_ref) + b1_ref[...])

    @pl.when(m == 3)
    def _m3():
        write(mm(x1_ref, w2_ref))


def _tc_prep_r(x1, x2, w0, w1, w2, b0, b1):
    return pl.pallas_call(
        _prep_r_body,
        grid=(NBLK, 4),
        in_specs=[
            pl.BlockSpec((RB, SUBS, CH), lambda i, m: (i, 0, 0)),
            pl.BlockSpec((RB, SUBS, CH), lambda i, m: (i, 0, 0)),
            pl.BlockSpec((SUBS, CH, D), lambda i, m: (0, 0, 0)),
            pl.BlockSpec((SUBS, CH, D), lambda i, m: (0, 0, 0)),
            pl.BlockSpec((SUBS, CH, D), lambda i, m: (0, 0, 0)),
            pl.BlockSpec((1, D), lambda i, m: (0, 0)),
            pl.BlockSpec((1, D), lambda i, m: (0, 0)),
        ],
        out_specs=pl.BlockSpec((RB, SUBS, CH),
                               lambda i, m: (m * NBLK + i, 0, 0)),
        out_shape=jax.ShapeDtypeStruct((4 * NPAD, SUBS, CH), _f32),
    )(x1, x2, w0, w1, w2, b0, b1)


_INV_SQRT_D = 1.0 / np.sqrt(float(D))


def _epi_common(p_ref, xaw_ref, att_ref):
    t = att_ref[0] - att_ref[1]
    att0 = jax.nn.sigmoid(jnp.full((RB, 1), t, _f32))
    att1 = 1.0 - att0
    deg0 = p_ref[2 * NMAT * SUBS, :, 0:1]
    deg1 = p_ref[2 * NMAT * SUBS + 1, :, 0:1]
    s0 = att0 / jnp.maximum(deg0, 1.0)
    s1 = att1 / jnp.maximum(deg1, 1.0)
    comb = [[s0 * p_ref[(0 * NMAT + mat) * SUBS + sub]
             + s1 * p_ref[(1 * NMAT + mat) * SUBS + sub]
             for sub in range(SUBS)] for mat in range(NMAT)]
    logits = jnp.zeros((RB, 1), _f32)
    for sub in range(SUBS):
        logits = logits + jnp.sum(comb[0][sub] * xaw_ref[0, :, sub, :],
                                  axis=1, keepdims=True)
    attw = jax.nn.sigmoid(logits * _INV_SQRT_D)
    return comb, attw


def _epi_a_body(p_ref, s2_ref, xaw_ref, att_ref, x1_ref, x2_ref):
    comb, attw = _epi_common(p_ref, xaw_ref, att_ref)
    for sub in range(SUBS):
        a1 = comb[1][sub]
        y1 = jnp.where(a1 >= 0, a1, 0.2 * a1)
        t2 = s2_ref[0, :, sub, :] - comb[2][sub]
        y2 = jnp.where(t2 >= 0, t2, 0.2 * t2)
        x1_ref[:, sub, :] = y1 + (1.0 - attw) * y2
        x2_ref[:, sub, :] = y2 + attw * y1


def _tc_epi_a(parts, catg4, att):
    return pl.pallas_call(
        _epi_a_body,
        grid=(NBLK,),
        in_specs=[
            pl.BlockSpec((NPART, RB, CH), lambda i: (0, i, 0)),
            pl.BlockSpec((1, RB, SUBS, CH), lambda i: (2, i, 0, 0)),
            pl.BlockSpec((1, RB, SUBS, CH), lambda i: (3, i, 0, 0)),
            pl.BlockSpec(memory_space=pltpu.SMEM),
        ],
        out_specs=[
            pl.BlockSpec((RB, SUBS, CH), lambda i: (i, 0, 0)),
            pl.BlockSpec((RB, SUBS, CH), lambda i: (i, 0, 0)),
        ],
        out_shape=[
            jax.ShapeDtypeStruct((NPAD, SUBS, CH), _f32),
            jax.ShapeDtypeStruct((NPAD, SUBS, CH), _f32),
        ],
    )(parts, catg4, catg4, att)


def _epi_r_body(p_ref, s2_ref, xaw_ref, att_ref, out_ref):
    comb, attw = _epi_common(p_ref, xaw_ref, att_ref)
    o = []
    ss = jnp.zeros((RB, 1), _f32)
    for sub in range(SUBS):
        o_sub = comb[1][sub] + (1.0 - attw) * (s2_ref[0, :, sub, :]
                                               - comb[2][sub])
        o.append(o_sub)
        ss = ss + jnp.sum(o_sub * o_sub, axis=1, keepdims=True)
    inv = 1.0 / jnp.maximum(jnp.sqrt(ss), 1e-12)
    for sub in range(SUBS):
        out_ref[:, sub * CH:(sub + 1) * CH] = o[sub] * inv


def _tc_epi_r(parts, catg4, att):
    return pl.pallas_call(
        _epi_r_body,
        grid=(NBLK,),
        in_specs=[
            pl.BlockSpec((NPART, RB, CH), lambda i: (0, i, 0)),
            pl.BlockSpec((1, RB, SUBS, CH), lambda i: (2, i, 0, 0)),
            pl.BlockSpec((1, RB, SUBS, CH), lambda i: (3, i, 0, 0)),
            pl.BlockSpec(memory_space=pltpu.SMEM),
        ],
        out_specs=pl.BlockSpec((RB, D), lambda i: (i, 0)),
        out_shape=jax.ShapeDtypeStruct((NPAD, D), _f32),
    )(parts, catg4, catg4, att)


def _sc_body(catv_hbm, src0_hbm, dst0_hbm, src1_hbm, dst1_hbm,
             p_hbm,
             sflat, dflat, c2ds, c2dd,
             zbuf_v, gidx_v, gbuf_v, obuf_v, gsem,
             acc_sh):
    cid = lax.axis_index("c")
    sid = lax.axis_index("s")

    start = jnp.minimum(sid * EPT, E - EPT)
    delta = sid * EPT - start
    ramp16 = lax.iota(_i32, 16)
    zero16f = jnp.zeros((16,), _f32)
    zero16i = jnp.zeros((16,), _i32)
    trash16 = jnp.full((16,), TRASH, _i32)
    ones16 = jnp.full((16,), 1.0, _f32)

    @pl.when(cid == 0)
    def _l0():
        pltpu.sync_copy(src0_hbm.at[pl.ds(start, EPT)], sflat)
        pltpu.sync_copy(dst0_hbm.at[pl.ds(start, EPT)], dflat)

    @pl.when(cid == 1)
    def _l1():
        pltpu.sync_copy(src1_hbm.at[pl.ds(start, EPT)], sflat)
        pltpu.sync_copy(dst1_hbm.at[pl.ds(start, EPT)], dflat)

    @pl.loop(0, ZQ_PT)
    def _zb(r):
        for c in range(CH // 16):
            zbuf_v[r, pl.ds(c * 16, 16)] = zero16f

    @pl.loop(0, EB)
    def _ob(r):
        for c in range(CH // 16):
            gbuf_v[2, r, pl.ds(c * 16, 16)] = ones16

    def zero_acc():
        pltpu.sync_copy(zbuf_v, acc_sh.at[pl.ds(sid * ZQ_PT, ZQ_PT)])

    def drain(prow, rows_pt):
        nd = rows_pt // 2
        for g in range(2):
            r = sid * rows_pt + g * nd
            pltpu.sync_copy(acc_sh.at[pl.ds(r, nd)], obuf_v.at[pl.ds(0, nd)])
            pltpu.sync_copy(obuf_v.at[pl.ds(0, nd)],
                            p_hbm.at[pl.ds(prow + r, nd)])

    @pl.loop(0, NRW)
    def _s2d(b):
        r = b // 8
        c = (b - r * 8) * 16
        c2ds[r, pl.ds(c, 16)] = sflat[pl.ds(b * 16, 16)]
    for k in range(NRW, NFB):
        r = k // 8
        c = (k - r * 8) * 16
        c2ds[r, pl.ds(c, 16)] = zero16i
        c2dd[r, pl.ds(c, 16)] = trash16

    for h in range(NQ):
        lo = h * QROWS

        @pl.loop(0, NRW)
        def _d2d(b, _lo=lo):
            dv = dflat[pl.ds(b * 16, 16)]
            pos = ramp16 + b * 16
            m = (pos >= delta) & (dv >= _lo) & (dv < _lo + QROWS)
            r = b // 8
            c = (b - r * 8) * 16
            c2dd[r, pl.ds(c, 16)] = jnp.where(m, dv - _lo, TRASH)

        nb = NB_E

        def edge_pass(gather_base, _nb=nb):
            def issue(j):
                p = j % 2
                for i in range(EB // 16):
                    sl = pl.ds(i * 16, 16)
                    gidx_v[p, sl] = c2ds[j, sl] * SUBS + gather_base
                pltpu.async_copy(catv_hbm.at[gidx_v.at[p]], gbuf_v.at[p],
                                 gsem)

            issue(0)

            @pl.loop(0, _nb)
            def _edges(j):
                @pl.when(j + 1 < _nb)
                def _nxt():
                    issue(j + 1)
                p = j % 2
                pltpu.make_async_copy(catv_hbm.at[gidx_v.at[p]],
                                      gbuf_v.at[p], gsem).wait()
                pltpu.sync_copy(gbuf_v.at[p], acc_sh.at[c2dd.at[j]],
                                add=True)

        rows_pt = min(QROWS, NPAD - lo) // NTILE

        @pl.loop(0, NMAT * SUBS)
        def _chunk(fc, _lo=lo, _ep=edge_pass, _rp=rows_pt):
            mat = fc // SUBS
            sub = fc - mat * SUBS
            zero_acc()
            plsc.subcore_barrier()
            _ep(mat * (NPAD * SUBS) + sub)
            plsc.subcore_barrier()
            drain(((cid * NMAT + mat) * SUBS + sub) * NPAD + _lo, _rp)

        zero_acc()
        plsc.subcore_barrier()

        @pl.loop(0, nb)
        def _deg(j):
            pltpu.sync_copy(gbuf_v.at[2], acc_sh.at[c2dd.at[j]], add=True)

        plsc.subcore_barrier()
        drain((2 * NMAT * SUBS + cid) * NPAD + lo, rows_pt)


def _sc_spmm(catv, src0, dst0, src1, dst1):
    mesh = plsc.VectorSubcoreMesh(core_axis_name="c", subcore_axis_name="s",
                                  num_cores=NCORE, num_subcores=NTILE)
    f = pl.kernel(
        _sc_body,
        out_type=jax.ShapeDtypeStruct((NPART * NPAD, CH), _f32),
        mesh=mesh,
        scratch_types=[
            pltpu.VMEM((EPT,), _i32),
            pltpu.VMEM((EPT,), _i32),
            pltpu.VMEM((NB_E, EB), _i32),
            pltpu.VMEM((NB_E, EB), _i32),
            pltpu.VMEM((ZQ_PT, CH), _f32),
            pltpu.VMEM((2, EB), _i32),
            pltpu.VMEM((3, EB, CH), _f32),
            pltpu.VMEM((112, CH), _f32),
            pltpu.SemaphoreType.DMA,
            pltpu.VMEM_SHARED((ACC_ROWS, CH), _f32),
        ],
    )
    return f(catv, src0, dst0, src1, dst1)


def kernel(x, edges0, edges1, Wc10, bc10, Wc1l, bc1l, Wc20, bc20, Wc2l, bc2l,
           AW0, AW1, a_att, r_att):
    e0c0 = edges0[:, 0]
    e0c1 = edges0[:, 1]
    e1c0 = edges1[:, 0]
    e1c1 = edges1[:, 1]

    catg_a = _tc_prep_a(x, Wc10, Wc20, AW0,
                        bc10.reshape(1, D), bc20.reshape(1, D))
    parts_a = _sc_spmm(catg_a.reshape(4 * NPAD * SUBS, CH),
                       e0c1, e0c0, e1c1, e1c0)
    x1, x2 = _tc_epi_a(parts_a.reshape(NPART, NPAD, CH),
                       catg_a.reshape(4, NPAD, SUBS, CH), a_att)

    catg_r = _tc_prep_r(x1, x2,
                        Wc1l.reshape(SUBS, CH, D), Wc2l.reshape(SUBS, CH, D),
                        AW1.reshape(SUBS, CH, D),
                        bc1l.reshape(1, D), bc2l.reshape(1, D))
    parts_r = _sc_spmm(catg_r.reshape(4 * NPAD * SUBS, CH),
                       e0c0, e0c1, e1c0, e1c1)
    out = _tc_epi_r(parts_r.reshape(NPART, NPAD, CH),
                    catg_r.reshape(4, NPAD, SUBS, CH), r_att)
    return out[:N]
